# Initial kernel scaffold; baseline (speedup 1.0000x reference)
import jax
import jax.numpy as jnp
from jax.experimental import pallas as pl


def kernel(y, edge_index, edge_weight, W1, b1, W2, b2, Wm, bm):
    raise NotImplementedError("write your pallas kernel here")


# speedup vs baseline: 8.8137x; 8.8137x over previous
import functools

import jax
import jax.numpy as jnp
from jax import lax
from jax.experimental import pallas as pl
from jax.experimental.pallas import tpu as pltpu
from jax.experimental.pallas import tpu_sc as plsc

N = 10000
F = 256
FH = 128
NC = 2
NS = 16
CHUNK = 128
CH = 84
E_PAD = NS * CH * CHUNK
RPT = 640
WB = 128

_mesh = plsc.VectorSubcoreMesh(
    core_axis_name="c", subcore_axis_name="s", num_cores=NC, num_subcores=NS
)


def _zero_bounce(bounce_v):
  zero = jnp.zeros((16,), jnp.float32)

  def body(i, _):
    for k in range(FH // 16):
      bounce_v[i, pl.ds(k * 16, 16)] = zero
    return 0

  lax.fori_loop(0, WB, body, 0)


DW = 8


def _deg_body(dst_hbm, w_hbm, deg_hbm, dst_v, w_v, rows_v, acc_sh):
  c = lax.axis_index("c")
  s = lax.axis_index("s")

  @pl.when(c == 0)
  def _():
    pltpu.sync_copy(dst_hbm.at[s], dst_v)
    pltpu.sync_copy(w_hbm.at[s], w_v)
    _zero_bounce(rows_v)
    base = pl.multiple_of(s * 640, 8)
    for i in range(640 // CHUNK):
      pltpu.sync_copy(rows_v, acc_sh.at[pl.ds(base + i * CHUNK, CHUNK)])
    plsc.subcore_barrier()

    def body(j, _):
      def fill(r16, _):
        wvec = w_v[j, pl.ds(r16 * 16, 16)]
        for rr in range(16):
          wv = jnp.full((16,), wvec[rr], jnp.float32)
          r = r16 * 16 + rr
          for k in range(FH // 16):
            rows_v[r, pl.ds(k * 16, 16)] = wv
        return 0

      lax.fori_loop(0, CHUNK // 16, fill, 0)
      pltpu.sync_copy(rows_v, acc_sh.at[dst_v.at[j]], add=True)
      return 0

    lax.fori_loop(0, CH, body, 0)
    plsc.subcore_barrier()

    for i in range(640 // CHUNK):
      pltpu.sync_copy(acc_sh.at[pl.ds(base + i * CHUNK, CHUNK)], rows_v)
      pltpu.sync_copy(rows_v, deg_hbm.at[pl.ds(base + i * CHUNK, CHUNK)])


_deg_call = pl.kernel(
    _deg_body,
    out_type=jax.ShapeDtypeStruct((N + 240, FH), jnp.float32),
    mesh=_mesh,
    scratch_types=[
        pltpu.VMEM((CH, CHUNK), jnp.int32),
        pltpu.VMEM((CH, CHUNK), jnp.float32),
        pltpu.VMEM((CHUNK, FH), jnp.float32),
        pltpu.VMEM_SHARED((N + 240, FH), jnp.float32),
    ],
)


def _edge_body(h_hbm, srcoff_hbm, dst_hbm, w_hbm, out_hbm,
               src_v, dst_v, w_v, rows_v, acc_sh, sem):
  c = lax.axis_index("c")
  s = lax.axis_index("s")

  pltpu.sync_copy(srcoff_hbm.at[c * NS + s], src_v)
  pltpu.sync_copy(dst_hbm.at[s], dst_v)
  pltpu.sync_copy(w_hbm.at[s], w_v)

  _zero_bounce(rows_v)
  base = pl.multiple_of(s * RPT, 8)

  @pl.when(s < NS - 1)
  def _():
    for i in range(RPT // WB):
      pltpu.sync_copy(rows_v, acc_sh.at[pl.ds(base + i * WB, WB)])

  @pl.when(s == NS - 1)
  def _():
    for i in range(3):
      pltpu.sync_copy(rows_v, acc_sh.at[pl.ds(base + i * WB, WB)])
    pltpu.sync_copy(rows_v.at[pl.ds(0, 16)],
                    acc_sh.at[pl.ds(base + 3 * WB, 16)])

  plsc.subcore_barrier()

  def chunk_body(j, _):
    pltpu.async_copy(h_hbm.at[src_v.at[j]], rows_v, sem).wait()

    def scale_body(r16, _):
      wvec = w_v[j, pl.ds(r16 * 16, 16)]
      for rr in range(16):
        wv = wvec[rr]
        r = r16 * 16 + rr
        for k in range(FH // 16):
          sl = pl.ds(k * 16, 16)
          rows_v[r, sl] = rows_v[r, sl] * wv
      return 0

    lax.fori_loop(0, CHUNK // 16, scale_body, 0)
    pltpu.sync_copy(rows_v, acc_sh.at[dst_v.at[j]], add=True)
    return 0

  lax.fori_loop(0, CH, chunk_body, 0)
  plsc.subcore_barrier()

  obase = pl.multiple_of(c * N + s * RPT, 8)

  def _wb(i, rows):
    pltpu.sync_copy(acc_sh.at[pl.ds(base + i * WB, rows)],
                    rows_v.at[pl.ds(0, rows)])
    pltpu.sync_copy(rows_v.at[pl.ds(0, rows)],
                    out_hbm.at[pl.ds(obase + i * WB, rows)])

  @pl.when(s < NS - 1)
  def _():
    for i in range(RPT // WB):
      _wb(i, WB)

  @pl.when(s == NS - 1)
  def _():
    for i in range(3):
      _wb(i, WB)
    _wb(3, 16)


_edge_call = pl.kernel(
    _edge_body,
    out_type=jax.ShapeDtypeStruct((NC * N, FH), jnp.float32),
    mesh=_mesh,
    scratch_types=[
        pltpu.VMEM((CH, CHUNK), jnp.int32),
        pltpu.VMEM((CH, CHUNK), jnp.int32),
        pltpu.VMEM((CH, CHUNK), jnp.float32),
        pltpu.VMEM((CHUNK, FH), jnp.float32),
        pltpu.VMEM_SHARED((N, FH), jnp.float32),
        pltpu.SemaphoreType.DMA,
    ],
)


RB = 2000
GR = N // RB


def _dis(deg_blk):
  d = jnp.maximum(deg_blk, 1e-12)
  return jnp.where(deg_blk > 0, lax.rsqrt(d), 0.0)


def _mm1_body(y_ref, w_ref, deg_ref, out_ref):
  dis = _dis(deg_ref[...])
  out_ref[...] = jnp.dot(
      y_ref[...], w_ref[...], preferred_element_type=jnp.float32) * dis


@jax.jit
def _mm1(y, W1, degc):
  return pl.pallas_call(
      _mm1_body,
      grid=(GR, NC),
      in_specs=[
          pl.BlockSpec((RB, F), lambda r, h: (r, 0)),
          pl.BlockSpec((F, FH), lambda r, h: (0, h)),
          pl.BlockSpec((RB, 1), lambda r, h: (r, 0)),
      ],
      out_specs=pl.BlockSpec((RB, FH), lambda r, h: (h * GR + r, 0)),
      out_shape=jax.ShapeDtypeStruct((NC * N, FH), jnp.float32),
  )(y, W1, degc)


def _mm2_body(aggt_ref, aggb_ref, deg_ref, b_ref, w_ref, out_ref):
  dis = _dis(deg_ref[...])
  x = jnp.concatenate([aggt_ref[...], aggb_ref[...]], axis=1)
  x = jnp.maximum(x * dis + b_ref[...], 0.0)
  out_ref[...] = jnp.dot(
      x, w_ref[...], preferred_element_type=jnp.float32) * dis


@jax.jit
def _mm2(agg, degc, b, W2):
  return pl.pallas_call(
      _mm2_body,
      grid=(GR, NC),
      in_specs=[
          pl.BlockSpec((RB, FH), lambda r, h: (r, 0)),
          pl.BlockSpec((RB, FH), lambda r, h: (GR + r, 0)),
          pl.BlockSpec((RB, 1), lambda r, h: (r, 0)),
          pl.BlockSpec((1, F), lambda r, h: (0, 0)),
          pl.BlockSpec((F, FH), lambda r, h: (0, h)),
      ],
      out_specs=pl.BlockSpec((RB, FH), lambda r, h: (h * GR + r, 0)),
      out_shape=jax.ShapeDtypeStruct((NC * N, FH), jnp.float32),
  )(agg, agg, degc, b.reshape(1, F), W2)


def _mm3_body(aggt_ref, aggb_ref, deg_ref, b_ref, w_ref, bm_ref, out_ref):
  dis = _dis(deg_ref[...])
  x = jnp.concatenate([aggt_ref[...], aggb_ref[...]], axis=1)
  x = jnp.maximum(x * dis + b_ref[...], 0.0)
  out_ref[...] = jnp.dot(
      x, w_ref[...], preferred_element_type=jnp.float32) + bm_ref[...]


@jax.jit
def _mm3(agg, degc, b, Wm, bm):
  return pl.pallas_call(
      _mm3_body,
      grid=(GR,),
      in_specs=[
          pl.BlockSpec((RB, FH), lambda r: (r, 0)),
          pl.BlockSpec((RB, FH), lambda r: (GR + r, 0)),
          pl.BlockSpec((RB, 1), lambda r: (r, 0)),
          pl.BlockSpec((1, F), lambda r: (0, 0)),
          pl.BlockSpec((F, FH), lambda r: (0, 0)),
          pl.BlockSpec((1, FH), lambda r: (0, 0)),
      ],
      out_specs=pl.BlockSpec((RB, FH), lambda r: (r, 0)),
      out_shape=jax.ShapeDtypeStruct((N, FH), jnp.float32),
  )(agg, agg, degc, b.reshape(1, F), Wm, bm.reshape(1, FH))


def kernel(y, edge_index, edge_weight, W1, b1, W2, b2, Wm, bm):
  src = edge_index[0].astype(jnp.int32)
  dst = edge_index[1].astype(jnp.int32)
  ew = edge_weight.astype(jnp.float32)

  loop = jnp.arange(N, dtype=jnp.int32)
  src_all = jnp.concatenate([src, loop])
  dst_all = jnp.concatenate([dst, loop])
  w_all = jnp.concatenate([ew, jnp.ones((N,), jnp.float32)])
  pad = E_PAD - src_all.shape[0]
  pad_idx = jnp.arange(pad, dtype=jnp.int32) % N
  src_all = jnp.concatenate([src_all, pad_idx])
  dst_all = jnp.concatenate([dst_all, pad_idx])
  w_all = jnp.concatenate([w_all, jnp.zeros((pad,), jnp.float32)])

  dst_t = dst_all.reshape(NS, CH, CHUNK)
  w_t = w_all.reshape(NS, CH, CHUNK)
  src_t = src_all.reshape(1, NS, CH, CHUNK)
  src_off = (src_t + (jnp.arange(NC, dtype=jnp.int32) * N)[:, None, None, None]
             ).reshape(NC * NS, CH, CHUNK)

  deg8 = _deg_call(dst_t, w_t)
  degc = deg8[:N, 0:1]

  h1 = _mm1(y, W1, degc)
  agg1 = _edge_call(h1, src_off, dst_t, w_t)
  h2 = _mm2(agg1, degc, b1, W2)
  agg2 = _edge_call(h2, src_off, dst_t, w_t)
  return _mm3(agg2, degc, b2, Wm, bm)


# speedup vs baseline: 13.2116x; 1.4990x over previous
import functools

import jax
import jax.numpy as jnp
from jax import lax
from jax.experimental import pallas as pl
from jax.experimental.pallas import tpu as pltpu
from jax.experimental.pallas import tpu_sc as plsc

N = 10000
F = 256
FH = 128
NC = 2
NS = 16
CHUNK = 128
CH = 84
E_PAD = NS * CH * CHUNK
RPT = 640
WB = 128

_mesh = plsc.VectorSubcoreMesh(
    core_axis_name="c", subcore_axis_name="s", num_cores=NC, num_subcores=NS
)


def _zero_bounce(bounce_v):
  zero = jnp.zeros((16,), jnp.float32)

  def body(i, _):
    for k in range(FH // 16):
      bounce_v[i, pl.ds(k * 16, 16)] = zero
    return 0

  lax.fori_loop(0, WB, body, 0)


DW = 8


def _deg_body(dst_hbm, w_hbm, deg_hbm, dst_v, w_v, rows_v, acc_sh):
  c = lax.axis_index("c")
  s = lax.axis_index("s")

  @pl.when(c == 0)
  def _():
    pltpu.sync_copy(dst_hbm.at[s], dst_v)
    pltpu.sync_copy(w_hbm.at[s], w_v)
    _zero_bounce(rows_v)
    base = pl.multiple_of(s * 640, 8)
    for i in range(640 // CHUNK):
      pltpu.sync_copy(rows_v, acc_sh.at[pl.ds(base + i * CHUNK, CHUNK)])
    plsc.subcore_barrier()

    def body(j, _):
      def fill(r16, _):
        wvec = w_v[j, pl.ds(r16 * 16, 16)]
        for rr in range(16):
          wv = jnp.full((16,), wvec[rr], jnp.float32)
          r = r16 * 16 + rr
          for k in range(FH // 16):
            rows_v[r, pl.ds(k * 16, 16)] = wv
        return 0

      lax.fori_loop(0, CHUNK // 16, fill, 0)
      pltpu.sync_copy(rows_v, acc_sh.at[dst_v.at[j]], add=True)
      return 0

    lax.fori_loop(0, CH, body, 0)
    plsc.subcore_barrier()

    for i in range(640 // CHUNK):
      pltpu.sync_copy(acc_sh.at[pl.ds(base + i * CHUNK, CHUNK)], rows_v)
      pltpu.sync_copy(rows_v, deg_hbm.at[pl.ds(base + i * CHUNK, CHUNK)])


_deg_call = pl.kernel(
    _deg_body,
    out_type=jax.ShapeDtypeStruct((N + 240, FH), jnp.float32),
    mesh=_mesh,
    scratch_types=[
        pltpu.VMEM((CH, CHUNK), jnp.int32),
        pltpu.VMEM((CH, CHUNK), jnp.float32),
        pltpu.VMEM((CHUNK, FH), jnp.float32),
        pltpu.VMEM_SHARED((N + 240, FH), jnp.float32),
    ],
)


NBUF = 3


def _edge_body(h_hbm, comb_hbm, wt_hbm, out_hbm,
               idx_v, wring_v, dstc_v, rows_v,
               sg0, sg1, sg2, ss0, ss1, ss2, si0, si1, si2, acc_sh):
  sem_g = (sg0, sg1, sg2)
  sem_s = (ss0, ss1, ss2)
  sem_i = (si0, si1, si2)
  c = lax.axis_index("c")
  s = lax.axis_index("s")
  tid = c * NS + s

  zero = jnp.zeros((16,), jnp.float32)

  def zb(i, _):
    for k in range(FH // 16):
      rows_v[0, i, pl.ds(k * 16, 16)] = zero
    return 0

  lax.fori_loop(0, WB, zb, 0)
  base = pl.multiple_of(s * RPT, 8)

  @pl.when(s < NS - 1)
  def _():
    for i in range(RPT // WB):
      pltpu.sync_copy(rows_v.at[0], acc_sh.at[pl.ds(base + i * WB, WB)])

  @pl.when(s == NS - 1)
  def _():
    for i in range(3):
      pltpu.sync_copy(rows_v.at[0], acc_sh.at[pl.ds(base + i * WB, WB)])
    pltpu.sync_copy(rows_v.at[0].at[pl.ds(0, 16)],
                    acc_sh.at[pl.ds(base + 3 * WB, 16)])

  pltpu.sync_copy(comb_hbm.at[tid].at[0], idx_v.at[0])
  pltpu.sync_copy(comb_hbm.at[tid].at[1], idx_v.at[1])
  pltpu.sync_copy(wt_hbm.at[s].at[0], wring_v.at[0])
  pltpu.sync_copy(wt_hbm.at[s].at[1], wring_v.at[1])
  pltpu.async_copy(h_hbm.at[idx_v.at[0].at[0]], rows_v.at[0], sem_g[0])
  plsc.subcore_barrier()

  def tri_body(jj, _):
    for b in range(NBUF):
      j = jj * NBUF + b
      bn = (b + 1) % NBUF

      @pl.when(j - 2 >= 0)
      def _():
        pltpu.make_async_copy(rows_v.at[bn], acc_sh.at[dstc_v.at[bn]],
                              sem_s[bn]).wait()

      @pl.when(jnp.logical_and(j >= 1, j + 1 < CH))
      def _():
        pltpu.make_async_copy(comb_hbm.at[tid].at[j + 1], idx_v.at[bn],
                              sem_i[bn]).wait()
        pltpu.make_async_copy(wt_hbm.at[s].at[j + 1], wring_v.at[bn],
                              sem_i[bn]).wait()

      @pl.when(j + 1 < CH)
      def _():
        pltpu.async_copy(h_hbm.at[idx_v.at[bn].at[0]], rows_v.at[bn],
                         sem_g[bn])

      @pl.when(j + 2 < CH)
      def _():
        bi = (b + 2) % NBUF
        pltpu.async_copy(comb_hbm.at[tid].at[j + 2], idx_v.at[bi],
                         sem_i[bi])
        pltpu.async_copy(wt_hbm.at[s].at[j + 2], wring_v.at[bi],
                         sem_i[bi])

      pltpu.make_async_copy(h_hbm.at[idx_v.at[b].at[0]], rows_v.at[b],
                            sem_g[b]).wait()

      def scale_body(r16, _):
        wvec = wring_v[b, pl.ds(r16 * 16, 16)]
        for rr in range(16):
          wv = wvec[rr]
          r = r16 * 16 + rr
          for k in range(FH // 16):
            sl = pl.ds(k * 16, 16)
            rows_v[b, r, sl] = rows_v[b, r, sl] * wv
        return 0

      lax.fori_loop(0, CHUNK // 16, scale_body, 0)
      for g in range(CHUNK // 16):
        dstc_v[b, pl.ds(g * 16, 16)] = idx_v[b, 1, pl.ds(g * 16, 16)]
      pltpu.async_copy(rows_v.at[b], acc_sh.at[dstc_v.at[b]], sem_s[b],
                       add=True)
    return 0

  lax.fori_loop(0, CH // NBUF, tri_body, 0)
  for j in (CH - 2, CH - 1):
    b = j % NBUF
    pltpu.make_async_copy(rows_v.at[b], acc_sh.at[dstc_v.at[b]],
                          sem_s[b]).wait()
  plsc.subcore_barrier()

  obase = pl.multiple_of(c * N + s * RPT, 8)

  def _wb(i, rows):
    pltpu.sync_copy(acc_sh.at[pl.ds(base + i * WB, rows)],
                    rows_v.at[0].at[pl.ds(0, rows)])
    pltpu.sync_copy(rows_v.at[0].at[pl.ds(0, rows)],
                    out_hbm.at[pl.ds(obase + i * WB, rows)])

  @pl.when(s < NS - 1)
  def _():
    for i in range(RPT // WB):
      _wb(i, WB)

  @pl.when(s == NS - 1)
  def _():
    for i in range(3):
      _wb(i, WB)
    _wb(3, 16)


_edge_call = pl.kernel(
    _edge_body,
    out_type=jax.ShapeDtypeStruct((NC * N, FH), jnp.float32),
    mesh=_mesh,
    scratch_types=[
        pltpu.VMEM((NBUF, 2, CHUNK), jnp.int32),
        pltpu.VMEM((NBUF, CHUNK), jnp.float32),
        pltpu.VMEM((NBUF, CHUNK), jnp.int32),
        pltpu.VMEM((NBUF, CHUNK, FH), jnp.float32),
        pltpu.SemaphoreType.DMA,
        pltpu.SemaphoreType.DMA,
        pltpu.SemaphoreType.DMA,
        pltpu.SemaphoreType.DMA,
        pltpu.SemaphoreType.DMA,
        pltpu.SemaphoreType.DMA,
        pltpu.SemaphoreType.DMA,
        pltpu.SemaphoreType.DMA,
        pltpu.SemaphoreType.DMA,
        pltpu.VMEM_SHARED((N, FH), jnp.float32),
    ],
)


RB = 2000
GR = N // RB


def _dis(deg_blk):
  d = jnp.maximum(deg_blk, 1e-12)
  return jnp.where(deg_blk > 0, lax.rsqrt(d), 0.0)


def _mm1_body(y_ref, w_ref, deg_ref, out_ref):
  dis = _dis(deg_ref[...])
  out_ref[...] = jnp.dot(
      y_ref[...], w_ref[...], preferred_element_type=jnp.float32) * dis


@jax.jit
def _mm1(y, W1, degc):
  return pl.pallas_call(
      _mm1_body,
      grid=(GR, NC),
      in_specs=[
          pl.BlockSpec((RB, F), lambda r, h: (r, 0)),
          pl.BlockSpec((F, FH), lambda r, h: (0, h)),
          pl.BlockSpec((RB, 1), lambda r, h: (r, 0)),
      ],
      out_specs=pl.BlockSpec((RB, FH), lambda r, h: (h * GR + r, 0)),
      out_shape=jax.ShapeDtypeStruct((NC * N, FH), jnp.float32),
  )(y, W1, degc)


def _mm2_body(aggt_ref, aggb_ref, deg_ref, b_ref, w_ref, out_ref):
  dis = _dis(deg_ref[...])
  x = jnp.concatenate([aggt_ref[...], aggb_ref[...]], axis=1)
  x = jnp.maximum(x * dis + b_ref[...], 0.0)
  out_ref[...] = jnp.dot(
      x, w_ref[...], preferred_element_type=jnp.float32) * dis


@jax.jit
def _mm2(agg, degc, b, W2):
  return pl.pallas_call(
      _mm2_body,
      grid=(GR, NC),
      in_specs=[
          pl.BlockSpec((RB, FH), lambda r, h: (r, 0)),
          pl.BlockSpec((RB, FH), lambda r, h: (GR + r, 0)),
          pl.BlockSpec((RB, 1), lambda r, h: (r, 0)),
          pl.BlockSpec((1, F), lambda r, h: (0, 0)),
          pl.BlockSpec((F, FH), lambda r, h: (0, h)),
      ],
      out_specs=pl.BlockSpec((RB, FH), lambda r, h: (h * GR + r, 0)),
      out_shape=jax.ShapeDtypeStruct((NC * N, FH), jnp.float32),
  )(agg, agg, degc, b.reshape(1, F), W2)


def _mm3_body(aggt_ref, aggb_ref, deg_ref, b_ref, w_ref, bm_ref, out_ref):
  dis = _dis(deg_ref[...])
  x = jnp.concatenate([aggt_ref[...], aggb_ref[...]], axis=1)
  x = jnp.maximum(x * dis + b_ref[...], 0.0)
  out_ref[...] = jnp.dot(
      x, w_ref[...], preferred_element_type=jnp.float32) + bm_ref[...]


@jax.jit
def _mm3(agg, degc, b, Wm, bm):
  return pl.pallas_call(
      _mm3_body,
      grid=(GR,),
      in_specs=[
          pl.BlockSpec((RB, FH), lambda r: (r, 0)),
          pl.BlockSpec((RB, FH), lambda r: (GR + r, 0)),
          pl.BlockSpec((RB, 1), lambda r: (r, 0)),
          pl.BlockSpec((1, F), lambda r: (0, 0)),
          pl.BlockSpec((F, FH), lambda r: (0, 0)),
          pl.BlockSpec((1, FH), lambda r: (0, 0)),
      ],
      out_specs=pl.BlockSpec((RB, FH), lambda r: (r, 0)),
      out_shape=jax.ShapeDtypeStruct((N, FH), jnp.float32),
  )(agg, agg, degc, b.reshape(1, F), Wm, bm.reshape(1, FH))


def kernel(y, edge_index, edge_weight, W1, b1, W2, b2, Wm, bm):
  src = edge_index[0].astype(jnp.int32)
  dst = edge_index[1].astype(jnp.int32)
  ew = edge_weight.astype(jnp.float32)

  loop = jnp.arange(N, dtype=jnp.int32)
  src_all = jnp.concatenate([src, loop])
  dst_all = jnp.concatenate([dst, loop])
  w_all = jnp.concatenate([ew, jnp.ones((N,), jnp.float32)])
  pad = E_PAD - src_all.shape[0]
  pad_idx = jnp.arange(pad, dtype=jnp.int32) % N
  src_all = jnp.concatenate([src_all, pad_idx])
  dst_all = jnp.concatenate([dst_all, pad_idx])
  w_all = jnp.concatenate([w_all, jnp.zeros((pad,), jnp.float32)])

  dst_t = dst_all.reshape(NS, CH, CHUNK)
  w_t = w_all.reshape(NS, CH, CHUNK)
  src_t = src_all.reshape(1, NS, CH, CHUNK)
  src_off = src_t + (jnp.arange(NC, dtype=jnp.int32) * N)[:, None, None, None]
  dst_b = jnp.broadcast_to(dst_t[None], (NC, NS, CH, CHUNK))
  comb = jnp.stack([src_off, dst_b], axis=3).reshape(NC * NS, CH, 2, CHUNK)

  deg8 = _deg_call(dst_t, w_t)
  degc = deg8[:N, 0:1]

  h1 = _mm1(y, W1, degc)
  agg1 = _edge_call(h1, comb, w_t)
  h2 = _mm2(agg1, degc, b1, W2)
  agg2 = _edge_call(h2, comb, w_t)
  return _mm3(agg2, degc, b2, Wm, bm)


# speedup vs baseline: 15.7950x; 1.1955x over previous
import jax
import jax.numpy as jnp
from jax import lax
from jax.experimental import pallas as pl
from jax.experimental.pallas import tpu as pltpu
from jax.experimental.pallas import tpu_sc as plsc

N = 10000
F = 256
FH = 128
NC = 2
NS = 16
CHUNK = 128
CH = 84
E_PAD = NS * CH * CHUNK
RPT = 640
WB = 128

_mesh = plsc.VectorSubcoreMesh(
    core_axis_name="c", subcore_axis_name="s", num_cores=NC, num_subcores=NS
)


def _zero_bounce(bounce_v):
  zero = jnp.zeros((16,), jnp.float32)

  def body(i, _):
    for k in range(FH // 16):
      bounce_v[i, pl.ds(k * 16, 16)] = zero
    return 0

  lax.fori_loop(0, WB, body, 0)


CHD = 42
NP = N + 240


def _deg_body(dst_hbm, w_hbm, deg_hbm, dst_v, w_v, rows_v, acc_sh):
  c = lax.axis_index("c")
  s = lax.axis_index("s")
  tid = c * NS + s

  pltpu.sync_copy(dst_hbm.at[tid], dst_v)
  pltpu.sync_copy(w_hbm.at[tid], w_v)
  _zero_bounce(rows_v)
  base = pl.multiple_of(s * 640, 8)
  for i in range(640 // CHUNK):
    pltpu.sync_copy(rows_v, acc_sh.at[pl.ds(base + i * CHUNK, CHUNK)])
  plsc.subcore_barrier()

  def body(j, _):
    @plsc.parallel_loop(0, CHUNK // 16, unroll=2)
    def _(r16):
      wvec = w_v[j, pl.ds(r16 * 16, 16)]
      for rr in range(16):
        rows_v[r16 * 16 + rr, pl.ds(0, 16)] = jnp.full(
            (16,), wvec[rr], jnp.float32)
    pltpu.sync_copy(rows_v, acc_sh.at[dst_v.at[j]], add=True)
    return 0

  lax.fori_loop(0, CHD, body, 0)
  plsc.subcore_barrier()

  obase = pl.multiple_of(c * NP + s * 640, 8)
  for i in range(640 // CHUNK):
    pltpu.sync_copy(acc_sh.at[pl.ds(base + i * CHUNK, CHUNK)], rows_v)
    pltpu.sync_copy(rows_v, deg_hbm.at[pl.ds(obase + i * CHUNK, CHUNK)])


_deg_call = pl.kernel(
    _deg_body,
    out_type=jax.ShapeDtypeStruct((NC * NP, FH), jnp.float32),
    mesh=_mesh,
    scratch_types=[
        pltpu.VMEM((CHD, CHUNK), jnp.int32),
        pltpu.VMEM((CHD, CHUNK), jnp.float32),
        pltpu.VMEM((CHUNK, FH), jnp.float32),
        pltpu.VMEM_SHARED((NP, FH), jnp.float32),
    ],
)


NBUF = 3


def _edge_body(h_hbm, comb_hbm, wt_hbm, out_hbm,
               idx_v, wring_v, dstc_v, rows_v,
               sg0, sg1, sg2, ss0, ss1, ss2, si0, si1, si2, acc_sh):
  sem_g = (sg0, sg1, sg2)
  sem_s = (ss0, ss1, ss2)
  sem_i = (si0, si1, si2)
  c = lax.axis_index("c")
  s = lax.axis_index("s")
  tid = c * NS + s

  zero = jnp.zeros((16,), jnp.float32)

  def zb(i, _):
    for k in range(FH // 16):
      rows_v[0, i, pl.ds(k * 16, 16)] = zero
    return 0

  lax.fori_loop(0, WB, zb, 0)
  base = pl.multiple_of(s * RPT, 8)

  @pl.when(s < NS - 1)
  def _():
    for i in range(RPT // WB):
      pltpu.sync_copy(rows_v.at[0], acc_sh.at[pl.ds(base + i * WB, WB)])

  @pl.when(s == NS - 1)
  def _():
    for i in range(3):
      pltpu.sync_copy(rows_v.at[0], acc_sh.at[pl.ds(base + i * WB, WB)])
    pltpu.sync_copy(rows_v.at[0].at[pl.ds(0, 16)],
                    acc_sh.at[pl.ds(base + 3 * WB, 16)])

  pltpu.sync_copy(comb_hbm.at[tid].at[0], idx_v.at[0])
  pltpu.sync_copy(comb_hbm.at[tid].at[1], idx_v.at[1])
  pltpu.sync_copy(wt_hbm.at[s].at[0], wring_v.at[0])
  pltpu.sync_copy(wt_hbm.at[s].at[1], wring_v.at[1])
  pltpu.async_copy(h_hbm.at[idx_v.at[0].at[0]], rows_v.at[0], sem_g[0])
  plsc.subcore_barrier()

  def tri_body(jj, _):
    for b in range(NBUF):
      j = jj * NBUF + b
      bn = (b + 1) % NBUF

      @pl.when(j - 2 >= 0)
      def _():
        pltpu.make_async_copy(rows_v.at[bn], acc_sh.at[dstc_v.at[bn]],
                              sem_s[bn]).wait()

      @pl.when(jnp.logical_and(j >= 1, j + 1 < CH))
      def _():
        pltpu.make_async_copy(comb_hbm.at[tid].at[j + 1], idx_v.at[bn],
                              sem_i[bn]).wait()
        pltpu.make_async_copy(wt_hbm.at[s].at[j + 1], wring_v.at[bn],
                              sem_i[bn]).wait()

      @pl.when(j + 1 < CH)
      def _():
        pltpu.async_copy(h_hbm.at[idx_v.at[bn].at[0]], rows_v.at[bn],
                         sem_g[bn])

      @pl.when(j + 2 < CH)
      def _():
        bi = (b + 2) % NBUF
        pltpu.async_copy(comb_hbm.at[tid].at[j + 2], idx_v.at[bi],
                         sem_i[bi])
        pltpu.async_copy(wt_hbm.at[s].at[j + 2], wring_v.at[bi],
                         sem_i[bi])

      pltpu.make_async_copy(h_hbm.at[idx_v.at[b].at[0]], rows_v.at[b],
                            sem_g[b]).wait()

      @plsc.parallel_loop(0, CHUNK // 16, unroll=2)
      def _(r16):
        wvec = wring_v[b, pl.ds(r16 * 16, 16)]
        for rr in range(16):
          wv = wvec[rr]
          r = r16 * 16 + rr
          for k in range(FH // 16):
            sl = pl.ds(k * 16, 16)
            rows_v[b, r, sl] = rows_v[b, r, sl] * wv
      for g in range(CHUNK // 16):
        dstc_v[b, pl.ds(g * 16, 16)] = idx_v[b, 1, pl.ds(g * 16, 16)]
      pltpu.async_copy(rows_v.at[b], acc_sh.at[dstc_v.at[b]], sem_s[b],
                       add=True)
    return 0

  lax.fori_loop(0, CH // NBUF, tri_body, 0)
  for j in (CH - 2, CH - 1):
    b = j % NBUF
    pltpu.make_async_copy(rows_v.at[b], acc_sh.at[dstc_v.at[b]],
                          sem_s[b]).wait()
  plsc.subcore_barrier()

  obase = pl.multiple_of(c * N + s * RPT, 8)

  def _wb(i, rows):
    pltpu.sync_copy(acc_sh.at[pl.ds(base + i * WB, rows)],
                    rows_v.at[0].at[pl.ds(0, rows)])
    pltpu.sync_copy(rows_v.at[0].at[pl.ds(0, rows)],
                    out_hbm.at[pl.ds(obase + i * WB, rows)])

  @pl.when(s < NS - 1)
  def _():
    for i in range(RPT // WB):
      _wb(i, WB)

  @pl.when(s == NS - 1)
  def _():
    for i in range(3):
      _wb(i, WB)
    _wb(3, 16)


_edge_call = pl.kernel(
    _edge_body,
    out_type=jax.ShapeDtypeStruct((NC * N, FH), jnp.float32),
    mesh=_mesh,
    scratch_types=[
        pltpu.VMEM((NBUF, 2, CHUNK), jnp.int32),
        pltpu.VMEM((NBUF, CHUNK), jnp.float32),
        pltpu.VMEM((NBUF, CHUNK), jnp.int32),
        pltpu.VMEM((NBUF, CHUNK, FH), jnp.float32),
        pltpu.SemaphoreType.DMA,
        pltpu.SemaphoreType.DMA,
        pltpu.SemaphoreType.DMA,
        pltpu.SemaphoreType.DMA,
        pltpu.SemaphoreType.DMA,
        pltpu.SemaphoreType.DMA,
        pltpu.SemaphoreType.DMA,
        pltpu.SemaphoreType.DMA,
        pltpu.SemaphoreType.DMA,
        pltpu.VMEM_SHARED((N, FH), jnp.float32),
    ],
)


RB = 2000
GR = N // RB


def _dis(deg_blk):
  d = jnp.maximum(deg_blk, 1e-12)
  return jnp.where(deg_blk > 0, lax.rsqrt(d), 0.0)


def _mm1_body(y_ref, w_ref, deg_ref, out_ref):
  dis = _dis(deg_ref[...])
  out_ref[...] = jnp.dot(
      y_ref[...], w_ref[...], preferred_element_type=jnp.float32) * dis


@jax.jit
def _mm1(y, W1, degc):
  return pl.pallas_call(
      _mm1_body,
      grid=(GR, NC),
      in_specs=[
          pl.BlockSpec((RB, F), lambda r, h: (r, 0)),
          pl.BlockSpec((F, FH), lambda r, h: (0, h)),
          pl.BlockSpec((RB, 1), lambda r, h: (r, 0)),
      ],
      out_specs=pl.BlockSpec((RB, FH), lambda r, h: (h * GR + r, 0)),
      out_shape=jax.ShapeDtypeStruct((NC * N, FH), jnp.float32),
  )(y, W1, degc)


def _mm2_body(aggt_ref, aggb_ref, deg_ref, b_ref, w_ref, out_ref):
  dis = _dis(deg_ref[...])
  x = jnp.concatenate([aggt_ref[...], aggb_ref[...]], axis=1)
  x = jnp.maximum(x * dis + b_ref[...], 0.0)
  out_ref[...] = jnp.dot(
      x, w_ref[...], preferred_element_type=jnp.float32) * dis


@jax.jit
def _mm2(agg, degc, b, W2):
  return pl.pallas_call(
      _mm2_body,
      grid=(GR, NC),
      in_specs=[
          pl.BlockSpec((RB, FH), lambda r, h: (r, 0)),
          pl.BlockSpec((RB, FH), lambda r, h: (GR + r, 0)),
          pl.BlockSpec((RB, 1), lambda r, h: (r, 0)),
          pl.BlockSpec((1, F), lambda r, h: (0, 0)),
          pl.BlockSpec((F, FH), lambda r, h: (0, h)),
      ],
      out_specs=pl.BlockSpec((RB, FH), lambda r, h: (h * GR + r, 0)),
      out_shape=jax.ShapeDtypeStruct((NC * N, FH), jnp.float32),
  )(agg, agg, degc, b.reshape(1, F), W2)


def _mm3_body(aggt_ref, aggb_ref, deg_ref, b_ref, w_ref, bm_ref, out_ref):
  dis = _dis(deg_ref[...])
  x = jnp.concatenate([aggt_ref[...], aggb_ref[...]], axis=1)
  x = jnp.maximum(x * dis + b_ref[...], 0.0)
  out_ref[...] = jnp.dot(
      x, w_ref[...], preferred_element_type=jnp.float32) + bm_ref[...]


@jax.jit
def _mm3(agg, degc, b, Wm, bm):
  return pl.pallas_call(
      _mm3_body,
      grid=(GR,),
      in_specs=[
          pl.BlockSpec((RB, FH), lambda r: (r, 0)),
          pl.BlockSpec((RB, FH), lambda r: (GR + r, 0)),
          pl.BlockSpec((RB, 1), lambda r: (r, 0)),
          pl.BlockSpec((1, F), lambda r: (0, 0)),
          pl.BlockSpec((F, FH), lambda r: (0, 0)),
          pl.BlockSpec((1, FH), lambda r: (0, 0)),
      ],
      out_specs=pl.BlockSpec((RB, FH), lambda r: (r, 0)),
      out_shape=jax.ShapeDtypeStruct((N, FH), jnp.float32),
  )(agg, agg, degc, b.reshape(1, F), Wm, bm.reshape(1, FH))


def kernel(y, edge_index, edge_weight, W1, b1, W2, b2, Wm, bm):
  src = edge_index[0].astype(jnp.int32)
  dst = edge_index[1].astype(jnp.int32)
  ew = edge_weight.astype(jnp.float32)

  loop = jnp.arange(N, dtype=jnp.int32)
  src_all = jnp.concatenate([src, loop])
  dst_all = jnp.concatenate([dst, loop])
  w_all = jnp.concatenate([ew, jnp.ones((N,), jnp.float32)])
  pad = E_PAD - src_all.shape[0]
  pad_idx = jnp.arange(pad, dtype=jnp.int32) % N
  src_all = jnp.concatenate([src_all, pad_idx])
  dst_all = jnp.concatenate([dst_all, pad_idx])
  w_all = jnp.concatenate([w_all, jnp.zeros((pad,), jnp.float32)])

  dst_t = dst_all.reshape(NS, CH, CHUNK)
  w_t = w_all.reshape(NS, CH, CHUNK)
  src_t = src_all.reshape(1, NS, CH, CHUNK)
  src_off = src_t + (jnp.arange(NC, dtype=jnp.int32) * N)[:, None, None, None]
  dst_b = jnp.broadcast_to(dst_t[None], (NC, NS, CH, CHUNK))
  comb = jnp.stack([src_off, dst_b], axis=3).reshape(NC * NS, CH, 2, CHUNK)

  dst_d = dst_all.reshape(NC * NS, CHD, CHUNK)
  w_d = w_all.reshape(NC * NS, CHD, CHUNK)
  deg8 = _deg_call(dst_d, w_d)
  degc = deg8[:N, 0:1] + deg8[NP:NP + N, 0:1]

  h1 = _mm1(y, W1, degc)
  agg1 = _edge_call(h1, comb, w_t)
  h2 = _mm2(agg1, degc, b1, W2)
  agg2 = _edge_call(h2, comb, w_t)
  return _mm3(agg2, degc, b2, Wm, bm)
